# initial kernel scaffold (unmeasured)
import functools

import jax
import jax.numpy as jnp
from jax import lax
from jax.experimental import pallas as pl
from jax.experimental.pallas import tpu as pltpu


def _mm_body(s_ref, x_ref, dy_ref, o_ref):
    k = pl.program_id(2)

    @pl.when(k == 0)
    def _():
        o_ref[...] = jnp.zeros_like(o_ref)

    o_ref[...] += lax.dot_general(
        x_ref[...],
        dy_ref[...],
        dimension_numbers=(((0,), (0,)), ((), ())),
        preferred_element_type=jnp.float32,
    )


def _matmul(s, x, dy, *, bn=1024, bk=512):
    K, Mx = x.shape
    N = dy.shape[1] // 2
    H = Mx // 2
    nj = N // bn
    grid_spec = pltpu.PrefetchScalarGridSpec(
        num_scalar_prefetch=1,
        grid=(2, nj, K // bk),
        in_specs=[
            pl.BlockSpec((bk, H), lambda i, j, k, s: (k, s[i])),
            pl.BlockSpec((bk, bn), lambda i, j, k, s: (k, s[2] * nj + j)),
        ],
        out_specs=pl.BlockSpec((H, bn), lambda i, j, k, s: (i, j)),
    )
    return pl.pallas_call(
        _mm_body,
        grid_spec=grid_spec,
        out_shape=jax.ShapeDtypeStruct((Mx, N), jnp.float32),
        compiler_params=pltpu.CompilerParams(
            dimension_semantics=("arbitrary", "arbitrary", "arbitrary"),
        ),
    )(s, x, dy)


def _swap(buf, *, axis, cid):

    def body(b_ref, r_ref, send_sem, recv_sem):
        my_x = lax.axis_index("x")
        my_y = lax.axis_index("y")
        if axis == "y":
            tgt = (my_x, 1 - my_y)
        else:
            tgt = (1 - my_x, my_y)
        barrier = pltpu.get_barrier_semaphore()
        pl.semaphore_signal(
            barrier, inc=1, device_id=tgt, device_id_type=pl.DeviceIdType.MESH
        )
        pl.semaphore_wait(barrier, 1)
        rdma = pltpu.make_async_remote_copy(
            src_ref=b_ref,
            dst_ref=r_ref,
            send_sem=send_sem,
            recv_sem=recv_sem,
            device_id=tgt,
            device_id_type=pl.DeviceIdType.MESH,
        )
        rdma.start()
        rdma.wait()

    return pl.pallas_call(
        body,
        out_shape=jax.ShapeDtypeStruct(buf.shape, buf.dtype),
        in_specs=[pl.BlockSpec(memory_space=pltpu.MemorySpace.ANY)],
        out_specs=pl.BlockSpec(memory_space=pltpu.MemorySpace.ANY),
        scratch_shapes=[pltpu.SemaphoreType.DMA, pltpu.SemaphoreType.DMA],
        compiler_params=pltpu.CompilerParams(
            collective_id=cid, has_side_effects=True
        ),
    )(buf)


def kernel(x, dy):
    my_x = lax.axis_index("x")
    my_y = lax.axis_index("y")
    s = jnp.stack([my_y, 1 - my_y, my_x]).astype(jnp.int32)

    H = x.shape[1] // 2

    Pp = _matmul(s, x, dy)
    recv_y = _swap(Pp[H:], axis="y", cid=0)
    O = Pp[:H] + recv_y
    recv_x = _swap(O, axis="x", cid=1)

    left = jnp.where(my_x == 0, O, recv_x)
    right = jnp.where(my_x == 0, recv_x, O)
    return jnp.concatenate([left, right], axis=1)


# baseline (device time: 1073083 ns/iter reference)
import functools

import jax
import jax.numpy as jnp
from jax import lax
from jax.experimental import pallas as pl
from jax.experimental.pallas import tpu as pltpu


def _mm_body(s_ref, x_ref, dy_ref, o_ref):
    k = pl.program_id(2)

    @pl.when(k == 0)
    def _():
        o_ref[...] = jnp.zeros_like(o_ref)

    o_ref[...] += lax.dot_general(
        x_ref[...],
        dy_ref[...],
        dimension_numbers=(((0,), (0,)), ((), ())),
        preferred_element_type=jnp.float32,
    )


def _matmul(s, x, dy, *, bn=1024, bk=512):
    K, Mx = x.shape
    N = dy.shape[1] // 2
    H = Mx // 2
    nj = N // bn
    grid_spec = pltpu.PrefetchScalarGridSpec(
        num_scalar_prefetch=1,
        grid=(2, nj, K // bk),
        in_specs=[
            pl.BlockSpec((bk, H), lambda i, j, k, s: (k, s[i])),
            pl.BlockSpec((bk, bn), lambda i, j, k, s: (k, s[2] * nj + j)),
        ],
        out_specs=pl.BlockSpec((H, bn), lambda i, j, k, s: (i, j)),
    )
    return pl.pallas_call(
        _mm_body,
        grid_spec=grid_spec,
        out_shape=jax.ShapeDtypeStruct((Mx, N), jnp.float32),
        compiler_params=pltpu.CompilerParams(
            dimension_semantics=("arbitrary", "arbitrary", "arbitrary"),
            vmem_limit_bytes=56 * 1024 * 1024,
        ),
    )(s, x, dy)


def _swap(buf, *, axis, cid):

    def body(b_ref, r_ref, send_sem, recv_sem):
        my_x = lax.axis_index("x")
        my_y = lax.axis_index("y")
        if axis == "y":
            tgt = (my_x, 1 - my_y)
        else:
            tgt = (1 - my_x, my_y)
        barrier = pltpu.get_barrier_semaphore()
        pl.semaphore_signal(
            barrier, inc=1, device_id=tgt, device_id_type=pl.DeviceIdType.MESH
        )
        pl.semaphore_wait(barrier, 1)
        rdma = pltpu.make_async_remote_copy(
            src_ref=b_ref,
            dst_ref=r_ref,
            send_sem=send_sem,
            recv_sem=recv_sem,
            device_id=tgt,
            device_id_type=pl.DeviceIdType.MESH,
        )
        rdma.start()
        rdma.wait()

    return pl.pallas_call(
        body,
        out_shape=jax.ShapeDtypeStruct(buf.shape, buf.dtype),
        in_specs=[pl.BlockSpec(memory_space=pltpu.MemorySpace.HBM)],
        out_specs=pl.BlockSpec(memory_space=pltpu.MemorySpace.HBM),
        scratch_shapes=[pltpu.SemaphoreType.DMA, pltpu.SemaphoreType.DMA],
        compiler_params=pltpu.CompilerParams(
            collective_id=cid, has_side_effects=True
        ),
    )(buf)


def kernel(x, dy):
    my_x = lax.axis_index("x")
    my_y = lax.axis_index("y")
    s = jnp.stack([my_y, 1 - my_y, my_x]).astype(jnp.int32)

    H = x.shape[1] // 2

    Pp = _matmul(s, x, dy)
    recv_y = _swap(Pp[H:], axis="y", cid=0)
    O = Pp[:H] + recv_y
    recv_x = _swap(O, axis="x", cid=1)

    left = jnp.where(my_x == 0, O, recv_x)
    right = jnp.where(my_x == 0, recv_x, O)
    return jnp.concatenate([left, right], axis=1)


# device time: 702453 ns/iter; 1.5276x vs baseline; 1.5276x over previous
import jax
import jax.numpy as jnp
from jax import lax
from jax.experimental import pallas as pl
from jax.experimental.pallas import tpu as pltpu

NC = 8
CN = 512
BK = 512


def kernel(x, dy):
    K, Mx = x.shape
    H = Mx // 2
    N = dy.shape[1] // 2
    NK = K // BK
    assert N == NC * CN

    my_x_outer = lax.axis_index("x")
    s = jnp.stack([my_x_outer]).astype(jnp.int32)

    def body(s_ref, x_ref, dy_ref, out_ref, stage_ref,
             pk_ref, ps_ref, o_ref, rv_ref,
             ysend, yrecv, xsend, xrecv, lsem):
        del s_ref
        c = pl.program_id(0)
        k = pl.program_id(1)
        my_x = lax.axis_index("x")
        my_y = lax.axis_index("y")
        y_tgt = (my_x, 1 - my_y)
        x_tgt = (1 - my_x, my_y)
        slot = lax.rem(c, 2)

        def y_rdma(cc, sl):
            return pltpu.make_async_remote_copy(
                src_ref=ps_ref.at[sl],
                dst_ref=stage_ref.at[cc],
                send_sem=ysend.at[cc],
                recv_sem=yrecv.at[cc],
                device_id=y_tgt,
                device_id_type=pl.DeviceIdType.MESH,
            )

        def x_rdma(cc, sl, col0):
            return pltpu.make_async_remote_copy(
                src_ref=o_ref.at[sl],
                dst_ref=out_ref.at[:, pl.ds(col0, CN)],
                send_sem=xsend.at[cc],
                recv_sem=xrecv.at[cc],
                device_id=x_tgt,
                device_id_type=pl.DeviceIdType.MESH,
            )

        @pl.when((c == 0) & (k == 0))
        def _barrier():
            bar = pltpu.get_barrier_semaphore()
            for tgt in (y_tgt, x_tgt):
                pl.semaphore_signal(
                    bar, inc=1, device_id=tgt,
                    device_id_type=pl.DeviceIdType.MESH,
                )
            pl.semaphore_wait(bar, 2)

        @pl.when((k == 0) & (c >= 2))
        def _wait_prev():
            y_rdma(c - 2, slot).wait_send()
            x_rdma(c - 2, slot, my_x * N).wait_send()
            pltpu.make_async_copy(
                o_ref.at[slot], out_ref.at[:, pl.ds(my_x * N, CN)],
                lsem.at[slot],
            ).wait()

        def add_phase(cm1):
            sl = lax.rem(cm1, 2)
            y_rdma(cm1, sl).wait_recv()
            ld = pltpu.make_async_copy(
                stage_ref.at[cm1], rv_ref.at[sl], lsem.at[sl]
            )
            ld.start()
            ld.wait()
            o_ref[sl] = pk_ref[sl] + rv_ref[sl]
            col0 = my_x * N + cm1 * CN
            pltpu.make_async_copy(
                o_ref.at[sl], out_ref.at[:, pl.ds(col0, CN)], lsem.at[sl]
            ).start()
            x_rdma(cm1, sl, col0).start()

        @pl.when((k == 0) & (c >= 1))
        def _add_mid():
            add_phase(c - 1)

        @pl.when(k == 0)
        def _zero():
            pk_ref[slot] = jnp.zeros((H, CN), jnp.float32)
            ps_ref[slot] = jnp.zeros((H, CN), jnp.float32)

        bb = dy_ref[...]
        a_keep = x_ref[:, pl.ds(my_y * H, H)]
        a_send = x_ref[:, pl.ds((1 - my_y) * H, H)]
        dn = (((0,), (0,)), ((), ()))
        pk_ref[slot] += lax.dot_general(
            a_keep, bb, dn, preferred_element_type=jnp.float32
        )
        ps_ref[slot] += lax.dot_general(
            a_send, bb, dn, preferred_element_type=jnp.float32
        )

        @pl.when(k == NK - 1)
        def _send_y():
            y_rdma(c, slot).start()

        @pl.when((c == NC - 1) & (k == NK - 1))
        def _final():
            add_phase(NC - 1)
            for cc in range(NC):
                x_rdma(cc, 0, 0).wait_recv()
            for cc in (NC - 2, NC - 1):
                y_rdma(cc, cc % 2).wait_send()
                x_rdma(cc, cc % 2, 0).wait_send()
            for sl in range(2):
                pltpu.make_async_copy(
                    o_ref.at[sl], out_ref.at[:, pl.ds(0, CN)], lsem.at[sl]
                ).wait()

    grid_spec = pltpu.PrefetchScalarGridSpec(
        num_scalar_prefetch=1,
        grid=(NC, NK),
        in_specs=[
            pl.BlockSpec((BK, Mx), lambda c, k, s: (k, 0)),
            pl.BlockSpec((BK, CN), lambda c, k, s: (k, s[0] * NC + c)),
        ],
        out_specs=(
            pl.BlockSpec(memory_space=pltpu.MemorySpace.HBM),
            pl.BlockSpec(memory_space=pltpu.MemorySpace.HBM),
        ),
        scratch_shapes=[
            pltpu.VMEM((2, H, CN), jnp.float32),
            pltpu.VMEM((2, H, CN), jnp.float32),
            pltpu.VMEM((2, H, CN), jnp.float32),
            pltpu.VMEM((2, H, CN), jnp.float32),
            pltpu.SemaphoreType.DMA((NC,)),
            pltpu.SemaphoreType.DMA((NC,)),
            pltpu.SemaphoreType.DMA((NC,)),
            pltpu.SemaphoreType.DMA((NC,)),
            pltpu.SemaphoreType.DMA((2,)),
        ],
    )
    out, _ = pl.pallas_call(
        body,
        grid_spec=grid_spec,
        out_shape=(
            jax.ShapeDtypeStruct((H, 2 * N), jnp.float32),
            jax.ShapeDtypeStruct((NC, H, CN), jnp.float32),
        ),
        compiler_params=pltpu.CompilerParams(
            dimension_semantics=("arbitrary", "arbitrary"),
            collective_id=0,
            has_side_effects=True,
            vmem_limit_bytes=60 * 1024 * 1024,
        ),
    )(s, x, dy)
    return out


# device time: 487569 ns/iter; 2.2009x vs baseline; 1.4407x over previous
import jax
import jax.numpy as jnp
from jax import lax
from jax.experimental import pallas as pl
from jax.experimental.pallas import tpu as pltpu

NC = 8
CN = 512
BK = 512


def kernel(x, dy):
    K, Mx = x.shape
    H = Mx // 2
    N = dy.shape[1] // 2
    NK = K // BK
    assert N == NC * CN

    my_x_outer = lax.axis_index("x")
    s = jnp.stack([my_x_outer]).astype(jnp.int32)

    def body(s_ref, x_ref, dy_ref, out_ref, stage_ref,
             pk_ref, ps_ref, o_ref, rv_ref,
             ysend, yrecv, xsend, xrecv, lsem):
        del s_ref
        c = pl.program_id(0)
        k = pl.program_id(1)
        my_x = lax.axis_index("x")
        my_y = lax.axis_index("y")
        y_tgt = (my_x, 1 - my_y)
        x_tgt = (1 - my_x, my_y)
        slot = lax.rem(c, 2)

        def y_rdma(cc, sl):
            return pltpu.make_async_remote_copy(
                src_ref=ps_ref.at[sl],
                dst_ref=stage_ref.at[cc],
                send_sem=ysend.at[cc],
                recv_sem=yrecv.at[cc],
                device_id=y_tgt,
                device_id_type=pl.DeviceIdType.MESH,
            )

        def x_rdma(cc, sl, col0):
            return pltpu.make_async_remote_copy(
                src_ref=o_ref.at[sl],
                dst_ref=out_ref.at[:, pl.ds(col0, CN)],
                send_sem=xsend.at[cc],
                recv_sem=xrecv.at[cc],
                device_id=x_tgt,
                device_id_type=pl.DeviceIdType.MESH,
            )

        @pl.when((c == 0) & (k == 0))
        def _barrier():
            bar = pltpu.get_barrier_semaphore()
            for tgt in (y_tgt, x_tgt):
                pl.semaphore_signal(
                    bar, inc=1, device_id=tgt,
                    device_id_type=pl.DeviceIdType.MESH,
                )
            pl.semaphore_wait(bar, 2)

        @pl.when((k == 0) & (c >= 2))
        def _wait_prev_y():
            y_rdma(c - 2, slot).wait_send()

        @pl.when((k == 0) & (c >= 3))
        def _wait_prev_x():
            x_rdma(c - 3, 1 - slot, my_x * N).wait_send()
            pltpu.make_async_copy(
                o_ref.at[1 - slot], out_ref.at[:, pl.ds(my_x * N, CN)],
                lsem.at[1 - slot],
            ).wait()

        def add_phase(cm1):
            sl = lax.rem(cm1, 2)
            y_rdma(cm1, sl).wait_recv()
            ld = pltpu.make_async_copy(
                stage_ref.at[cm1], rv_ref.at[sl], lsem.at[sl]
            )
            ld.start()
            ld.wait()
            o_ref[sl] = pk_ref[sl] + rv_ref[sl]
            col0 = my_x * N + cm1 * CN
            pltpu.make_async_copy(
                o_ref.at[sl], out_ref.at[:, pl.ds(col0, CN)], lsem.at[sl]
            ).start()
            x_rdma(cm1, sl, col0).start()

        @pl.when(k == 0)
        def _zero():
            pk_ref[slot] = jnp.zeros((H, CN), jnp.float32)
            ps_ref[slot] = jnp.zeros((H, CN), jnp.float32)

        bb = dy_ref[...]
        a_keep = x_ref[:, pl.ds(my_y * H, H)]
        a_send = x_ref[:, pl.ds((1 - my_y) * H, H)]
        dn = (((0,), (0,)), ((), ()))
        pk_ref[slot] += lax.dot_general(
            a_keep, bb, dn, preferred_element_type=jnp.float32
        )
        ps_ref[slot] += lax.dot_general(
            a_send, bb, dn, preferred_element_type=jnp.float32
        )

        @pl.when(k == NK - 1)
        def _send_y():
            y_rdma(c, slot).start()

        @pl.when((k == NK - 1) & (c >= 1))
        def _add_mid():
            add_phase(c - 1)

        @pl.when((c == NC - 1) & (k == NK - 1))
        def _final():
            sl_last = (NC - 1) % 2
            x_rdma(NC - 3, sl_last, 0).wait_send()
            pltpu.make_async_copy(
                o_ref.at[sl_last], out_ref.at[:, pl.ds(0, CN)],
                lsem.at[sl_last],
            ).wait()
            add_phase(NC - 1)
            for cc in range(NC):
                x_rdma(cc, 0, 0).wait_recv()
            for cc in (NC - 2, NC - 1):
                y_rdma(cc, cc % 2).wait_send()
                x_rdma(cc, cc % 2, 0).wait_send()
            for sl in (0, 1):
                pltpu.make_async_copy(
                    o_ref.at[sl], out_ref.at[:, pl.ds(0, CN)], lsem.at[sl]
                ).wait()

    grid_spec = pltpu.PrefetchScalarGridSpec(
        num_scalar_prefetch=1,
        grid=(NC, NK),
        in_specs=[
            pl.BlockSpec((BK, Mx), lambda c, k, s: (k, 0)),
            pl.BlockSpec((BK, CN), lambda c, k, s: (k, s[0] * NC + c)),
        ],
        out_specs=(
            pl.BlockSpec(memory_space=pltpu.MemorySpace.HBM),
            pl.BlockSpec(memory_space=pltpu.MemorySpace.HBM),
        ),
        scratch_shapes=[
            pltpu.VMEM((2, H, CN), jnp.float32),
            pltpu.VMEM((2, H, CN), jnp.float32),
            pltpu.VMEM((2, H, CN), jnp.float32),
            pltpu.VMEM((2, H, CN), jnp.float32),
            pltpu.SemaphoreType.DMA((NC,)),
            pltpu.SemaphoreType.DMA((NC,)),
            pltpu.SemaphoreType.DMA((NC,)),
            pltpu.SemaphoreType.DMA((NC,)),
            pltpu.SemaphoreType.DMA((2,)),
        ],
    )
    out, _ = pl.pallas_call(
        body,
        grid_spec=grid_spec,
        out_shape=(
            jax.ShapeDtypeStruct((H, 2 * N), jnp.float32),
            jax.ShapeDtypeStruct((NC, H, CN), jnp.float32),
        ),
        compiler_params=pltpu.CompilerParams(
            dimension_semantics=("arbitrary", "arbitrary"),
            collective_id=0,
            has_side_effects=True,
            vmem_limit_bytes=60 * 1024 * 1024,
        ),
    )(s, x, dy)
    return out
